# baseline (speedup 1.0000x reference)
import functools

import jax
import jax.numpy as jnp
from jax.experimental import pallas as pl

N, E, D, A_, H, SG = 10000, 160000, 128, 3, 128, 16
NP = 10240
BLK = 640
NBLK = E // BLK


def _lk(x):
    return jnp.where(x > 0, x, 0.01 * x)


def _dense_body(x_ref, w_ref, b_ref, o_ref, *, act):
    y = jnp.dot(x_ref[...], w_ref[...], preferred_element_type=jnp.float32)
    y = y + b_ref[...]
    o_ref[...] = _lk(y) if act else y


def _dense(x, w, b, act, blk=BLK):
    r, k = x.shape
    n = w.shape[1]
    return pl.pallas_call(
        functools.partial(_dense_body, act=act),
        grid=(r // blk,),
        in_specs=[
            pl.BlockSpec((blk, k), lambda i: (i, 0)),
            pl.BlockSpec((k, n), lambda i: (0, 0)),
            pl.BlockSpec((1, n), lambda i: (0, 0)),
        ],
        out_specs=pl.BlockSpec((blk, n), lambda i: (i, 0)),
        out_shape=jax.ShapeDtypeStruct((r, n), jnp.float32),
    )(x, w, b)


def _h_body(pg_ref, qg_ref, ef_ref, w_ref, b_ref, o_ref):
    y = pg_ref[...] + qg_ref[...] + jnp.dot(
        ef_ref[...], w_ref[...], preferred_element_type=jnp.float32) + b_ref[...]
    o_ref[...] = _lk(y)


def _h_stage(pg, qg, ef, w1c, b1):
    return pl.pallas_call(
        _h_body,
        grid=(NBLK,),
        in_specs=[
            pl.BlockSpec((BLK, D), lambda i: (i, 0)),
            pl.BlockSpec((BLK, D), lambda i: (i, 0)),
            pl.BlockSpec((BLK, A_ + 1), lambda i: (i, 0)),
            pl.BlockSpec((A_ + 1, D), lambda i: (0, 0)),
            pl.BlockSpec((1, D), lambda i: (0, 0)),
        ],
        out_specs=pl.BlockSpec((BLK, D), lambda i: (i, 0)),
        out_shape=jax.ShapeDtypeStruct((E, D), jnp.float32),
    )(pg, qg, ef, w1c, b1)


def _e_body(rg_ref, sg_ref, h_ref, w3c_ref, b3_ref, ws_ref, gwa_ref, gwb_ref,
            gt_ref, a_ref, b_ref, sp_ref):
    e = _lk(rg_ref[...] + sg_ref[...] + jnp.dot(
        h_ref[...], w3c_ref[...], preferred_element_type=jnp.float32)
        + b3_ref[...])
    logits = jnp.dot(e, ws_ref[...], preferred_element_type=jnp.float32)
    sig = 1.0 / (1.0 + jnp.exp(-logits))
    dif = sig - gt_ref[...]
    sp_ref[...] = jnp.sum(dif * dif).reshape(1, 1, 1)
    a_ref[...] = jnp.dot(e, gwa_ref[...], preferred_element_type=jnp.float32)
    b_ref[...] = jnp.dot(e, gwb_ref[...], preferred_element_type=jnp.float32)


def _e_stage(rg, sg, h, w3c, b3, ws, gwa, gwb, gt2):
    return pl.pallas_call(
        _e_body,
        grid=(NBLK,),
        in_specs=[
            pl.BlockSpec((BLK, D), lambda i: (i, 0)),
            pl.BlockSpec((BLK, D), lambda i: (i, 0)),
            pl.BlockSpec((BLK, D), lambda i: (i, 0)),
            pl.BlockSpec((D, D), lambda i: (0, 0)),
            pl.BlockSpec((1, D), lambda i: (0, 0)),
            pl.BlockSpec((D, 1), lambda i: (0, 0)),
            pl.BlockSpec((D, D), lambda i: (0, 0)),
            pl.BlockSpec((D, D), lambda i: (0, 0)),
            pl.BlockSpec((BLK, 1), lambda i: (i, 0)),
        ],
        out_specs=[
            pl.BlockSpec((BLK, D), lambda i: (i, 0)),
            pl.BlockSpec((BLK, D), lambda i: (i, 0)),
            pl.BlockSpec((1, 1, 1), lambda i: (i, 0, 0)),
        ],
        out_shape=[
            jax.ShapeDtypeStruct((E, D), jnp.float32),
            jax.ShapeDtypeStruct((E, D), jnp.float32),
            jax.ShapeDtypeStruct((NBLK, 1, 1), jnp.float32),
        ],
    )(rg, sg, h, w3c, b3, ws, gwa, gwb, gt2)


def _f_body(ag_ref, sc_ref, gb_ref, o_ref):
    f = _lk(ag_ref[...] + sc_ref[...] + gb_ref[...])
    o_ref[...] = jnp.mean(f.reshape(BLK // SG, SG, D), axis=1)


def _f_stage(ag, scat, gb):
    return pl.pallas_call(
        _f_body,
        grid=(NBLK,),
        in_specs=[
            pl.BlockSpec((BLK, D), lambda i: (i, 0)),
            pl.BlockSpec((BLK, D), lambda i: (i, 0)),
            pl.BlockSpec((1, D), lambda i: (0, 0)),
        ],
        out_specs=pl.BlockSpec((BLK // SG, D), lambda i: (i, 0)),
        out_shape=jax.ShapeDtypeStruct((N, D), jnp.float32),
    )(ag, scat, gb)


def _bn(x, g, b):
    m = jnp.mean(x, axis=0, keepdims=True)
    xc = x - m
    v = jnp.mean(xc * xc, axis=0, keepdims=True)
    return xc / jnp.sqrt(v + 1e-5) * g + b


def _vhead(x, w):
    x = _lk(_bn(x, w[0], w[1]))
    x = jnp.dot(x, w[2], preferred_element_type=jnp.float32) + w[3]
    x = _lk(_bn(x, w[4], w[5]))
    x = jnp.dot(x, w[6], preferred_element_type=jnp.float32) + w[7]
    x = _lk(_bn(x, w[8], w[9]))
    return jnp.dot(x, w[10], preferred_element_type=jnp.float32) + w[11]


def _v_body(*refs):
    x1, x2 = refs[0][...], refs[1][...]
    w1 = [r[...] for r in refs[2:14]]
    w2 = [r[...] for r in refs[14:26]]
    refs[26][...] = _vhead(x1, w1)
    refs[27][...] = _vhead(x2, w2)


def _v_stage(p1, p2, w1, w2):
    return pl.pallas_call(
        _v_body,
        out_shape=[jax.ShapeDtypeStruct((N, 1), jnp.float32)] * 2,
    )(p1, p2, *w1, *w2)


def _row2(v):
    return v.reshape(1, -1)


def kernel(node_features, actions, edge_index, angles, sub_graphs, sep_subgraphs, gt_edges, post_input, q1_W1, q1_b1, q1_W2, q1_b2, q1_W3, q1_b3, q1_ws, q2_W1, q2_b1, q2_W2, q2_b2, q2_W3, q2_b3, q2_ws, g1_W, g1_b, g2_W, g2_b, v1_g0, v1_be0, v1_W1, v1_b1, v1_g1, v1_be1, v1_W2, v1_b2, v1_g2, v1_be2, v1_W3, v1_b3, v2_g0, v2_be0, v2_W1, v2_b1, v2_g1, v2_be1, v2_W2, v2_b2, v2_g2, v2_be2, v2_W3, v2_b3):
    ef = jnp.concatenate([actions, angles], axis=1)
    src, dst = edge_index[0], edge_index[1]
    sub_idx = sub_graphs[0]
    sep = sep_subgraphs[0]
    sg0 = jnp.concatenate([sep[0], sep[1]])
    sg1 = jnp.concatenate([sep[1], sep[0]])
    gt2 = gt_edges.reshape(E, 1)
    nf_p = jnp.pad(node_features, ((0, NP - N), (0, 0)))

    def one_q(W1, b1, W2, b2, W3, b3, ws, gW, gb):
        PQ = _dense(nf_p, jnp.concatenate([W1[:D], W1[D:2 * D]], axis=1),
                    jnp.zeros((1, 2 * D), jnp.float32), False)
        P, Q = PQ[:, :D], PQ[:, D:]
        h = _h_stage(P[src], Q[dst], ef, W1[2 * D:], _row2(b1))
        agg = jax.ops.segment_sum(h, dst, num_segments=N)
        agg_p = jnp.pad(agg, ((0, NP - N), (0, 0)))
        nu = _dense(agg_p, W2, _row2(b2), True)
        RS = _dense(nu, jnp.concatenate([W3[:D], W3[D:2 * D]], axis=1),
                    jnp.zeros((1, 2 * D), jnp.float32), False)
        R, S = RS[:, :D], RS[:, D:]
        A, B, sp = _e_stage(R[src], S[dst], h, W3[2 * D:], _row2(b3), ws,
                            gW[:D], gW[D:], gt2)
        side_q = jnp.sum(sp) / E
        Bg = B[sub_idx]
        scat = jax.ops.segment_sum(Bg[sg0], sg1, num_segments=E)
        pooled = _f_stage(A[sub_idx], scat, _row2(gb))
        return pooled, side_q

    p1, s1 = one_q(q1_W1, q1_b1, q1_W2, q1_b2, q1_W3, q1_b3, q1_ws, g1_W, g1_b)
    p2, s2 = one_q(q2_W1, q2_b1, q2_W2, q2_b2, q2_W3, q2_b3, q2_ws, g2_W, g2_b)

    w1 = [_row2(v1_g0), _row2(v1_be0), v1_W1, _row2(v1_b1), _row2(v1_g1),
          _row2(v1_be1), v1_W2, _row2(v1_b2), _row2(v1_g2), _row2(v1_be2),
          v1_W3, _row2(v1_b3)]
    w2 = [_row2(v2_g0), _row2(v2_be0), v2_W1, _row2(v2_b1), _row2(v2_g1),
          _row2(v2_be1), v2_W2, _row2(v2_b2), _row2(v2_g2), _row2(v2_be2),
          v2_W3, _row2(v2_b3)]
    q1, q2 = _v_stage(p1, p2, w1, w2)

    side = (s1 + s2 + 0.0 * post_input) / 4.0
    return q1[:, 0], q2[:, 0], side


# speedup vs baseline: 1.0586x; 1.0586x over previous
import functools

import jax
import jax.numpy as jnp
from jax.experimental import pallas as pl

N, E, D, A_, H, SG = 10000, 160000, 128, 3, 128, 16
NP = 10240
BLK = 640
NBLK = E // BLK


def _lk(x):
    return jnp.where(x > 0, x, 0.01 * x)


def _dense_body(x_ref, w_ref, b_ref, o_ref, *, act):
    y = jnp.dot(x_ref[...], w_ref[...], preferred_element_type=jnp.float32)
    y = y + b_ref[...]
    o_ref[...] = _lk(y) if act else y


def _dense(x, w, b, act, blk=BLK):
    r, k = x.shape
    n = w.shape[1]
    return pl.pallas_call(
        functools.partial(_dense_body, act=act),
        grid=(r // blk,),
        in_specs=[
            pl.BlockSpec((blk, k), lambda i: (i, 0)),
            pl.BlockSpec((k, n), lambda i: (0, 0)),
            pl.BlockSpec((1, n), lambda i: (0, 0)),
        ],
        out_specs=pl.BlockSpec((blk, n), lambda i: (i, 0)),
        out_shape=jax.ShapeDtypeStruct((r, n), jnp.float32),
    )(x, w, b)


def _h_body(pg_ref, qg_ref, ef_ref, w_ref, b_ref, o_ref):
    y = pg_ref[...] + qg_ref[...] + jnp.dot(
        ef_ref[...], w_ref[...], preferred_element_type=jnp.float32) + b_ref[...]
    o_ref[...] = _lk(y)


def _h_stage(pg, qg, ef, w1c, b1):
    w = pg.shape[1]
    return pl.pallas_call(
        _h_body,
        grid=(NBLK,),
        in_specs=[
            pl.BlockSpec((BLK, w), lambda i: (i, 0)),
            pl.BlockSpec((BLK, w), lambda i: (i, 0)),
            pl.BlockSpec((BLK, A_ + 1), lambda i: (i, 0)),
            pl.BlockSpec((A_ + 1, w), lambda i: (0, 0)),
            pl.BlockSpec((1, w), lambda i: (0, 0)),
        ],
        out_specs=pl.BlockSpec((BLK, w), lambda i: (i, 0)),
        out_shape=jax.ShapeDtypeStruct((E, w), jnp.float32),
    )(pg, qg, ef, w1c, b1)


def _e_body(rg_ref, sg_ref, h_ref, w3c1_ref, w3c2_ref, b3_ref, ws1_ref,
            ws2_ref, gwa1_ref, gwa2_ref, gwb1_ref, gwb2_ref, gt_ref,
            a_ref, b_ref, sp1_ref, sp2_ref):
    x = rg_ref[...] + sg_ref[...] + b3_ref[...]
    h = h_ref[...]
    gt = gt_ref[...]
    for sl, w3c, ws, gwa, gwb, sp in (
            (slice(0, D), w3c1_ref, ws1_ref, gwa1_ref, gwb1_ref, sp1_ref),
            (slice(D, 2 * D), w3c2_ref, ws2_ref, gwa2_ref, gwb2_ref, sp2_ref)):
        e = _lk(x[:, sl] + jnp.dot(h[:, sl], w3c[...],
                                   preferred_element_type=jnp.float32))
        logits = jnp.dot(e, ws[...], preferred_element_type=jnp.float32)
        sig = 1.0 / (1.0 + jnp.exp(-logits))
        dif = sig - gt
        sp[...] = jnp.sum(dif * dif).reshape(1, 1, 1)
        a_ref[:, sl] = jnp.dot(e, gwa[...], preferred_element_type=jnp.float32)
        b_ref[:, sl] = jnp.dot(e, gwb[...], preferred_element_type=jnp.float32)


def _e_stage(rg, sg, h, w3c1, w3c2, b3, ws1, ws2, gwa1, gwa2, gwb1, gwb2, gt2):
    wmat = lambda a, b: pl.BlockSpec((a, b), lambda i: (0, 0))
    return pl.pallas_call(
        _e_body,
        grid=(NBLK,),
        in_specs=[
            pl.BlockSpec((BLK, 2 * D), lambda i: (i, 0)),
            pl.BlockSpec((BLK, 2 * D), lambda i: (i, 0)),
            pl.BlockSpec((BLK, 2 * D), lambda i: (i, 0)),
            wmat(D, D), wmat(D, D), wmat(1, 2 * D), wmat(D, 1), wmat(D, 1),
            wmat(D, D), wmat(D, D), wmat(D, D), wmat(D, D),
            pl.BlockSpec((BLK, 1), lambda i: (i, 0)),
        ],
        out_specs=[
            pl.BlockSpec((BLK, 2 * D), lambda i: (i, 0)),
            pl.BlockSpec((BLK, 2 * D), lambda i: (i, 0)),
            pl.BlockSpec((1, 1, 1), lambda i: (i, 0, 0)),
            pl.BlockSpec((1, 1, 1), lambda i: (i, 0, 0)),
        ],
        out_shape=[
            jax.ShapeDtypeStruct((E, 2 * D), jnp.float32),
            jax.ShapeDtypeStruct((E, 2 * D), jnp.float32),
            jax.ShapeDtypeStruct((NBLK, 1, 1), jnp.float32),
            jax.ShapeDtypeStruct((NBLK, 1, 1), jnp.float32),
        ],
    )(rg, sg, h, w3c1, w3c2, b3, ws1, ws2, gwa1, gwa2, gwb1, gwb2, gt2)


def _f_body(ag_ref, sc_ref, gb_ref, o_ref):
    f = _lk(ag_ref[...] + sc_ref[...] + gb_ref[...])
    o_ref[...] = jnp.mean(f.reshape(BLK // SG, SG, 2 * D), axis=1)


def _f_stage(ag, scat, gb):
    return pl.pallas_call(
        _f_body,
        grid=(NBLK,),
        in_specs=[
            pl.BlockSpec((BLK, 2 * D), lambda i: (i, 0)),
            pl.BlockSpec((BLK, 2 * D), lambda i: (i, 0)),
            pl.BlockSpec((1, 2 * D), lambda i: (0, 0)),
        ],
        out_specs=pl.BlockSpec((BLK // SG, 2 * D), lambda i: (i, 0)),
        out_shape=jax.ShapeDtypeStruct((N, 2 * D), jnp.float32),
    )(ag, scat, gb)


def _bn(x, g, b):
    m = jnp.mean(x, axis=0, keepdims=True)
    xc = x - m
    v = jnp.mean(xc * xc, axis=0, keepdims=True)
    return xc / jnp.sqrt(v + 1e-5) * g + b


def _vhead(x, w):
    x = _lk(_bn(x, w[0], w[1]))
    x = jnp.dot(x, w[2], preferred_element_type=jnp.float32) + w[3]
    x = _lk(_bn(x, w[4], w[5]))
    x = jnp.dot(x, w[6], preferred_element_type=jnp.float32) + w[7]
    x = _lk(_bn(x, w[8], w[9]))
    return jnp.dot(x, w[10], preferred_element_type=jnp.float32) + w[11]


def _v_body(*refs):
    x1, x2 = refs[0][...], refs[1][...]
    w1 = [r[...] for r in refs[2:14]]
    w2 = [r[...] for r in refs[14:26]]
    refs[26][...] = _vhead(x1, w1)
    refs[27][...] = _vhead(x2, w2)


def _v_stage(p1, p2, w1, w2):
    return pl.pallas_call(
        _v_body,
        out_shape=[jax.ShapeDtypeStruct((N, 1), jnp.float32)] * 2,
    )(p1, p2, *w1, *w2)


def _row2(v):
    return v.reshape(1, -1)


def kernel(node_features, actions, edge_index, angles, sub_graphs, sep_subgraphs, gt_edges, post_input, q1_W1, q1_b1, q1_W2, q1_b2, q1_W3, q1_b3, q1_ws, q2_W1, q2_b1, q2_W2, q2_b2, q2_W3, q2_b3, q2_ws, g1_W, g1_b, g2_W, g2_b, v1_g0, v1_be0, v1_W1, v1_b1, v1_g1, v1_be1, v1_W2, v1_b2, v1_g2, v1_be2, v1_W3, v1_b3, v2_g0, v2_be0, v2_W1, v2_b1, v2_g1, v2_be1, v2_W2, v2_b2, v2_g2, v2_be2, v2_W3, v2_b3):
    ef = jnp.concatenate([actions, angles], axis=1)
    src, dst = edge_index[0], edge_index[1]
    sub_idx = sub_graphs[0]
    sep = sep_subgraphs[0]
    sg0 = jnp.concatenate([sep[0], sep[1]])
    sg1 = jnp.concatenate([sep[1], sep[0]])
    gt2 = gt_edges.reshape(E, 1)
    nf_p = jnp.pad(node_features, ((0, NP - N), (0, 0)))

    zz = jnp.zeros((D, D), jnp.float32)
    bdiag = lambda a, b: jnp.block([[a, zz], [zz, b]])

    Pt = _dense(nf_p, jnp.concatenate([q1_W1[:D], q2_W1[:D]], axis=1),
                jnp.zeros((1, 2 * D), jnp.float32), False)
    Qt = _dense(nf_p, jnp.concatenate([q1_W1[D:2 * D], q2_W1[D:2 * D]], axis=1),
                jnp.zeros((1, 2 * D), jnp.float32), False)
    h12 = _h_stage(Pt[src], Qt[dst], ef,
                   jnp.concatenate([q1_W1[2 * D:], q2_W1[2 * D:]], axis=1),
                   jnp.concatenate([q1_b1, q2_b1]).reshape(1, 2 * D))
    agg = jax.ops.segment_sum(h12, dst, num_segments=N)
    agg_p = jnp.pad(agg, ((0, NP - N), (0, 0)))
    nu12 = _dense(agg_p, bdiag(q1_W2, q2_W2),
                  jnp.concatenate([q1_b2, q2_b2]).reshape(1, 2 * D), True)
    Rt = _dense(nu12, bdiag(q1_W3[:D], q2_W3[:D]),
                jnp.zeros((1, 2 * D), jnp.float32), False)
    St = _dense(nu12, bdiag(q1_W3[D:2 * D], q2_W3[D:2 * D]),
                jnp.zeros((1, 2 * D), jnp.float32), False)
    A12, B12, sp1, sp2 = _e_stage(
        Rt[src], St[dst], h12, q1_W3[2 * D:], q2_W3[2 * D:],
        jnp.concatenate([q1_b3, q2_b3]).reshape(1, 2 * D), q1_ws, q2_ws,
        g1_W[:D], g2_W[:D], g1_W[D:], g2_W[D:], gt2)
    s1 = jnp.sum(sp1) / E
    s2 = jnp.sum(sp2) / E
    idxc = sub_idx[sg0]
    scat = jax.ops.segment_sum(B12[idxc], sg1, num_segments=E)
    pooled = _f_stage(A12[sub_idx], scat,
                      jnp.concatenate([g1_b, g2_b]).reshape(1, 2 * D))
    p1, p2 = pooled[:, :D], pooled[:, D:]

    w1 = [_row2(v1_g0), _row2(v1_be0), v1_W1, _row2(v1_b1), _row2(v1_g1),
          _row2(v1_be1), v1_W2, _row2(v1_b2), _row2(v1_g2), _row2(v1_be2),
          v1_W3, _row2(v1_b3)]
    w2 = [_row2(v2_g0), _row2(v2_be0), v2_W1, _row2(v2_b1), _row2(v2_g1),
          _row2(v2_be1), v2_W2, _row2(v2_b2), _row2(v2_g2), _row2(v2_be2),
          v2_W3, _row2(v2_b3)]
    q1, q2 = _v_stage(p1, p2, w1, w2)

    side = (s1 + s2 + 0.0 * post_input) / 4.0
    return q1[:, 0], q2[:, 0], side
